# baseline (speedup 1.0000x reference)
import jax
import jax.numpy as jnp
from jax import lax
from jax.experimental import pallas as pl
from jax.experimental.pallas import tpu as pltpu
from jax.experimental.pallas import tpu_sc as plsc

N = 10000
F_IN = 128
H = 128
EMB = 64
E = 320000

NC, NS = 2, 16
NW = NC * NS
CHUNK = 64
EPT = 10240
NCHUNK = EPT // CHUNK
EP = NW * EPT
NACC = 10112
RPT = NACC // NS
DRAIN = (128, 128, 128, 128, 120)

_MESH = dict(core_axis_name="c", subcore_axis_name="s")

BN_EPS = 1e-5
R = 2000
GB = N // R


NP = 10240
SPT = NP // NS
DCHUNK = 128
DNCHUNK = EPT // DCHUNK


def _sc_deg_body(dst_hbm, w_hbm, out_hbm, dst_v, w_v, stage_v, acc_sh):
    cid = lax.axis_index("c")
    sid = lax.axis_index("s")
    wid = cid * NS + sid

    def z_body(i, c):
        stage_v[pl.ds(i * 16, 16)] = jnp.zeros((16,), jnp.float32)
        return c

    lax.fori_loop(0, SPT // 16, z_body, 0)
    pltpu.sync_copy(stage_v, acc_sh.at[pl.ds(sid * SPT, SPT)])
    pltpu.sync_copy(dst_hbm.at[wid], dst_v)
    pltpu.sync_copy(w_hbm.at[wid], w_v)
    plsc.subcore_barrier()

    def chunk_body(j, c):
        pltpu.sync_copy(w_v.at[j], acc_sh.at[dst_v.at[j]], add=True)
        return c

    lax.fori_loop(0, DNCHUNK, chunk_body, 0)
    plsc.subcore_barrier()
    pltpu.sync_copy(acc_sh.at[pl.ds(sid * SPT, SPT)], stage_v)
    pltpu.sync_copy(stage_v, out_hbm.at[pl.ds(cid * NP + sid * SPT, SPT)])


def _run_deg(dst_p3, w_p3):
    call = pl.kernel(
        _sc_deg_body,
        out_type=jax.ShapeDtypeStruct((NC * NP,), jnp.float32),
        mesh=plsc.VectorSubcoreMesh(**_MESH),
        scratch_types=[
            pltpu.VMEM((DNCHUNK, DCHUNK), jnp.int32),
            pltpu.VMEM((DNCHUNK, DCHUNK), jnp.float32),
            pltpu.VMEM((SPT,), jnp.float32),
            pltpu.VMEM_SHARED((NP,), jnp.float32),
        ],
    )
    return call(dst_p3, w_p3)


def _sc_scatter_body(u_hbm, src_hbm, dst_hbm, w_hbm, out_hbm,
                     src_v, gbuf0, gbuf1, sbuf0, sbuf1,
                     dbuf0, dbuf1, dbuf2, dbuf3, wbuf0, wbuf1,
                     acc_sh, sg0, sg1, sd0, sd1, ss0, ss1):
    cid = lax.axis_index("c")
    sid = lax.axis_index("s")
    wid = cid * NS + sid
    gbufs, sgs = (gbuf0, gbuf1), (sg0, sg1)
    sbufs, sss = (sbuf0, sbuf1), (ss0, ss1)
    dbufs = (dbuf0, dbuf1, dbuf2, dbuf3)
    wbufs, sds = (wbuf0, wbuf1), (sd0, sd1)

    def z_body(i, c):
        for cc in range(H // 16):
            gbuf0[i, pl.ds(cc * 16, 16)] = jnp.zeros((16,), jnp.float32)
        return c

    lax.fori_loop(0, CHUNK, z_body, 0)
    off = 0
    for sz in DRAIN:
        for piece in range((sz + CHUNK - 1) // CHUNK):
            psz = min(CHUNK, sz - piece * CHUNK)
            pltpu.sync_copy(
                gbuf0.at[pl.ds(0, psz)],
                acc_sh.at[pl.ds(sid * RPT + off + piece * CHUNK, psz)])
        off += sz
    pltpu.sync_copy(src_hbm.at[wid], src_v)
    plsc.subcore_barrier()

    for b in range(2):
        pltpu.async_copy(u_hbm.at[src_v.at[pl.ds(b * CHUNK, CHUNK)]],
                         gbufs[b], sgs[b])
        pltpu.async_copy(dst_hbm.at[wid, pl.ds(b * CHUNK, CHUNK)],
                         dbufs[b], sds[b])
        pltpu.async_copy(w_hbm.at[wid, pl.ds(b * CHUNK, CHUNK)],
                         wbufs[b], sds[b])

    def quad_body(q, carry):
        for b in range(4):
            j = 4 * q + b
            gb, sg = gbufs[b % 2], sgs[b % 2]
            sb, ss = sbufs[b % 2], sss[b % 2]
            wb, sd = wbufs[b % 2], sds[b % 2]
            db = dbufs[b]
            db_pre = dbufs[(b + 2) % 4]
            csl = pl.ds(j * CHUNK, CHUNK)

            pltpu.make_async_copy(u_hbm.at[src_v.at[csl]], gb, sg).wait()
            pltpu.make_async_copy(dst_hbm.at[wid, csl], db, sd).wait()
            pltpu.make_async_copy(w_hbm.at[wid, csl], wb, sd).wait()

            @pl.when(j >= 2)
            def _():
                pltpu.make_async_copy(sb, acc_sh.at[db], ss).wait()

            def grp_body(g, c2):
                base = g * 16
                w16 = wb[pl.ds(base, 16)]
                for r in range(16):
                    w = w16[r]
                    row = base + r
                    for c in range(H // 16):
                        sl = pl.ds(c * 16, 16)
                        sb[row, sl] = gb[row, sl] * w
                return c2

            lax.fori_loop(0, CHUNK // 16, grp_body, 0)
            pltpu.async_copy(sb, acc_sh.at[db], ss, add=True)

            @pl.when(j + 2 < NCHUNK)
            def _():
                psl = pl.ds((j + 2) * CHUNK, CHUNK)
                pltpu.async_copy(u_hbm.at[src_v.at[psl]], gb, sg)
                pltpu.async_copy(dst_hbm.at[wid, psl], db_pre, sd)
                pltpu.async_copy(w_hbm.at[wid, psl], wb, sd)
        return carry

    lax.fori_loop(0, NCHUNK // 4, quad_body, 0)
    pltpu.make_async_copy(sbuf0, acc_sh.at[dbuf2], ss0).wait()
    pltpu.make_async_copy(sbuf1, acc_sh.at[dbuf3], ss1).wait()
    plsc.subcore_barrier()
    off = 0
    for sz in DRAIN:
        for piece in range((sz + CHUNK - 1) // CHUNK):
            psz = min(CHUNK, sz - piece * CHUNK)
            po = sid * RPT + off + piece * CHUNK
            pltpu.sync_copy(acc_sh.at[pl.ds(po, psz)],
                            gbuf0.at[pl.ds(0, psz)])
            pltpu.sync_copy(gbuf0.at[pl.ds(0, psz)],
                            out_hbm.at[cid, pl.ds(po, psz)])
        off += sz


def _run_scatter(u, src_p, dst_p, w_p):
    call = pl.kernel(
        _sc_scatter_body,
        out_type=jax.ShapeDtypeStruct((NC, NACC, H), jnp.float32),
        mesh=plsc.VectorSubcoreMesh(**_MESH),
        scratch_types=[
            pltpu.VMEM((EPT,), jnp.int32),
            pltpu.VMEM((CHUNK, H), jnp.float32),
            pltpu.VMEM((CHUNK, H), jnp.float32),
            pltpu.VMEM((CHUNK, H), jnp.float32),
            pltpu.VMEM((CHUNK, H), jnp.float32),
            pltpu.VMEM((CHUNK,), jnp.int32),
            pltpu.VMEM((CHUNK,), jnp.int32),
            pltpu.VMEM((CHUNK,), jnp.int32),
            pltpu.VMEM((CHUNK,), jnp.int32),
            pltpu.VMEM((CHUNK,), jnp.float32),
            pltpu.VMEM((CHUNK,), jnp.float32),
            pltpu.VMEM_SHARED((NACC, H), jnp.float32),
            pltpu.SemaphoreType.DMA,
            pltpu.SemaphoreType.DMA,
            pltpu.SemaphoreType.DMA,
            pltpu.SemaphoreType.DMA,
            pltpu.SemaphoreType.DMA,
            pltpu.SemaphoreType.DMA,
        ],
    )
    return call(u, src_p, dst_p, w_p)


def _dinv_body(d2_ref, dinv_ref):
    deg = jnp.sum(d2_ref[...], axis=0, keepdims=True) + 1.0
    dinv_ref[...] = lax.rsqrt(deg)


def _run_dinv(d2):
    return pl.pallas_call(
        _dinv_body,
        out_shape=jax.ShapeDtypeStruct((1, N), jnp.float32),
    )(d2)


def _t0_body(x_ref, winT_ref, bin_ref, w1T_ref, dinv_ref, a_ref, u_ref):
    h = jnp.dot(x_ref[...], winT_ref[...],
                preferred_element_type=jnp.float32) + bin_ref[...]
    a = jnp.dot(h, w1T_ref[...], preferred_element_type=jnp.float32)
    a_ref[...] = a
    u_ref[...] = a * dinv_ref[...]


def _run_t0(x, winT, bin_, w1T, dinv2):
    return pl.pallas_call(
        _t0_body,
        grid=(GB,),
        in_specs=[
            pl.BlockSpec((R, F_IN), lambda i: (i, 0)),
            pl.BlockSpec((F_IN, H), lambda i: (0, 0)),
            pl.BlockSpec((1, H), lambda i: (0, 0)),
            pl.BlockSpec((H, H), lambda i: (0, 0)),
            pl.BlockSpec((R, 1), lambda i: (i, 0)),
        ],
        out_specs=[
            pl.BlockSpec((R, H), lambda i: (i, 0)),
            pl.BlockSpec((R, H), lambda i: (i, 0)),
        ],
        out_shape=[
            jax.ShapeDtypeStruct((N, H), jnp.float32),
            jax.ShapeDtypeStruct((N, H), jnp.float32),
        ],
    )(x, winT, bin_, w1T, dinv2)


def _ts_body(p_ref, a_ref, b_ref, dinv_ref, out_ref, st_ref):
    ps = p_ref[0] + p_ref[1]
    dv = dinv_ref[...]
    ob = ps * dv + a_ref[...] * (dv * dv) + b_ref[...]
    out_ref[...] = ob

    @pl.when(pl.program_id(0) == 0)
    def _():
        st_ref[...] = jnp.zeros_like(st_ref)

    st_ref[...] += jnp.concatenate(
        [jnp.sum(ob, 0, keepdims=True), jnp.sum(ob * ob, 0, keepdims=True)],
        axis=0)


def _run_ts(p, a, b, dinv2):
    return pl.pallas_call(
        _ts_body,
        grid=(GB,),
        in_specs=[
            pl.BlockSpec((NC, R, H), lambda i: (0, i, 0)),
            pl.BlockSpec((R, H), lambda i: (i, 0)),
            pl.BlockSpec((1, H), lambda i: (0, 0)),
            pl.BlockSpec((R, 1), lambda i: (i, 0)),
        ],
        out_specs=[
            pl.BlockSpec((R, H), lambda i: (i, 0)),
            pl.BlockSpec((2, H), lambda i: (0, 0)),
        ],
        out_shape=[
            jax.ShapeDtypeStruct((N, H), jnp.float32),
            jax.ShapeDtypeStruct((2, H), jnp.float32),
        ],
    )(p, a, b, dinv2)


def _bn_relu(o, st, g, be):
    mean = st[0:1, :] * (1.0 / N)
    var = st[1:2, :] * (1.0 / N) - mean * mean
    return jnp.maximum((o - mean) * lax.rsqrt(var + BN_EPS) * g + be, 0.0)


def _ta_body(o_ref, st_ref, g_ref, be_ref, wT_ref, dinv_ref, a2_ref, u2_ref):
    hb = _bn_relu(o_ref[...], st_ref[...], g_ref[...], be_ref[...])
    a2 = jnp.dot(hb, wT_ref[...], preferred_element_type=jnp.float32)
    a2_ref[...] = a2
    u2_ref[...] = a2 * dinv_ref[...]


def _run_ta(o, st, g, be, wT, dinv2):
    return pl.pallas_call(
        _ta_body,
        grid=(GB,),
        in_specs=[
            pl.BlockSpec((R, H), lambda i: (i, 0)),
            pl.BlockSpec((2, H), lambda i: (0, 0)),
            pl.BlockSpec((1, H), lambda i: (0, 0)),
            pl.BlockSpec((1, H), lambda i: (0, 0)),
            pl.BlockSpec((H, H), lambda i: (0, 0)),
            pl.BlockSpec((R, 1), lambda i: (i, 0)),
        ],
        out_specs=[
            pl.BlockSpec((R, H), lambda i: (i, 0)),
            pl.BlockSpec((R, H), lambda i: (i, 0)),
        ],
        out_shape=[
            jax.ShapeDtypeStruct((N, H), jnp.float32),
            jax.ShapeDtypeStruct((N, H), jnp.float32),
        ],
    )(o, st, g, be, wT, dinv2)


def _t3_body(o_ref, st_ref, g_ref, be_ref, wn1T_ref, bn1_ref, wn2T_ref,
             bn2_ref, wc1T_ref, bc1_ref, wc2T_ref, bc2_ref,
             h_ref, node_ref, clust_ref, nsum_ref):
    hb = _bn_relu(o_ref[...], st_ref[...], g_ref[...], be_ref[...])
    h_ref[...] = hb
    z = jnp.maximum(
        jnp.dot(hb, wn1T_ref[...], preferred_element_type=jnp.float32)
        + bn1_ref[...], 0.0)
    node = jnp.dot(z, wn2T_ref[...],
                   preferred_element_type=jnp.float32) + bn2_ref[...]
    node_ref[...] = node
    c = jnp.maximum(
        jnp.dot(node, wc1T_ref[...], preferred_element_type=jnp.float32)
        + bc1_ref[...], 0.0)
    clust_ref[...] = jnp.dot(c, wc2T_ref[...],
                             preferred_element_type=jnp.float32) + bc2_ref[...]

    @pl.when(pl.program_id(0) == 0)
    def _():
        nsum_ref[...] = jnp.zeros_like(nsum_ref)

    nsum_ref[...] += jnp.sum(node, 0, keepdims=True)


def _run_t3(o, st, g, be, wn1T, bn1, wn2T, bn2, wc1T, bc1, wc2T, bc2):
    full = lambda r, c: pl.BlockSpec((r, c), lambda i: (0, 0))
    return pl.pallas_call(
        _t3_body,
        grid=(GB,),
        in_specs=[
            pl.BlockSpec((R, H), lambda i: (i, 0)),
            full(2, H), full(1, H), full(1, H),
            full(H, EMB), full(1, EMB),
            full(EMB, EMB), full(1, EMB),
            full(EMB, EMB), full(1, EMB),
            full(EMB, EMB // 2), full(1, EMB // 2),
        ],
        out_specs=[
            pl.BlockSpec((R, H), lambda i: (i, 0)),
            pl.BlockSpec((R, EMB), lambda i: (i, 0)),
            pl.BlockSpec((R, EMB // 2), lambda i: (i, 0)),
            pl.BlockSpec((1, EMB), lambda i: (0, 0)),
        ],
        out_shape=[
            jax.ShapeDtypeStruct((N, H), jnp.float32),
            jax.ShapeDtypeStruct((N, EMB), jnp.float32),
            jax.ShapeDtypeStruct((N, EMB // 2), jnp.float32),
            jax.ShapeDtypeStruct((1, EMB), jnp.float32),
        ],
    )(o, st, g, be, wn1T, bn1, wn2T, bn2, wc1T, bc1, wc2T, bc2)


def _t4_body(nsum_ref, wg1T_ref, bg1_ref, wg2T_ref, bg2_ref, graph_ref):
    m = nsum_ref[...] * (1.0 / N)
    gv = jnp.maximum(
        jnp.dot(m, wg1T_ref[...], preferred_element_type=jnp.float32)
        + bg1_ref[...], 0.0)
    graph_ref[...] = jnp.dot(gv, wg2T_ref[...],
                             preferred_element_type=jnp.float32) + bg2_ref[...]


def _run_t4(nsum, wg1T, bg1, wg2T, bg2):
    return pl.pallas_call(
        _t4_body,
        out_shape=jax.ShapeDtypeStruct((1, EMB), jnp.float32),
    )(nsum, wg1T, bg1, wg2T, bg2)


def kernel(x, edge_index, edge_weight, params):
    src = edge_index[0]
    dst = edge_index[1]
    pad = EP - E
    fill = jnp.arange(pad, dtype=src.dtype) % N
    src_p = jnp.concatenate([src, fill]).reshape(NW, EPT)
    dst_full = jnp.concatenate([dst, fill])
    w_full = jnp.concatenate([edge_weight, jnp.zeros((pad,), edge_weight.dtype)])
    dst_p = dst_full.reshape(NW, EPT)
    w_p = w_full.reshape(NW, EPT)
    dst_p3 = dst_full.reshape(NW, DNCHUNK, DCHUNK)
    w_p3 = w_full.reshape(NW, DNCHUNK, DCHUNK)

    winT = params['in_proj'][0].T
    bin_ = params['in_proj'][1].reshape(1, H)
    gcn = params['gcn']
    wT = [l['Wb'][0].T for l in gcn]
    bs = [l['Wb'][1].reshape(1, H) for l in gcn]
    gs = [l['gamma'].reshape(1, H) for l in gcn]
    bes = [l['beta'].reshape(1, H) for l in gcn]
    wn1T = params['node_emb'][0][0].T
    bn1 = params['node_emb'][0][1].reshape(1, EMB)
    wn2T = params['node_emb'][1][0].T
    bn2 = params['node_emb'][1][1].reshape(1, EMB)
    wg1T = params['graph_emb'][0][0].T
    bg1 = params['graph_emb'][0][1].reshape(1, H)
    wg2T = params['graph_emb'][1][0].T
    bg2 = params['graph_emb'][1][1].reshape(1, EMB)
    wc1T = params['clust'][0][0].T
    bc1 = params['clust'][0][1].reshape(1, EMB)
    wc2T = params['clust'][1][0].T
    bc2 = params['clust'][1][1].reshape(1, EMB // 2)

    dflat = _run_deg(dst_p3, w_p3)
    d2 = dflat.reshape(NC, NP)[:, :N]
    dinv = _run_dinv(d2)
    dinv2 = dinv.reshape(N, 1)

    a, u = _run_t0(x, winT, bin_, wT[0], dinv2)
    for i in range(3):
        p = _run_scatter(u, src_p, dst_p, w_p)
        o, st = _run_ts(p, a, bs[i], dinv2)
        if i < 2:
            a, u = _run_ta(o, st, gs[i], bes[i], wT[i + 1], dinv2)
    h, node, clust, nsum = _run_t3(o, st, gs[2], bes[2],
                                   wn1T, bn1, wn2T, bn2,
                                   wc1T, bc1, wc2T, bc2)
    graph = _run_t4(nsum, wg1T, bg1, wg2T, bg2)
    return (node, graph, clust, h)


# speedup vs baseline: 1.4626x; 1.4626x over previous
import jax
import jax.numpy as jnp
from jax import lax
from jax.experimental import pallas as pl
from jax.experimental.pallas import tpu as pltpu
from jax.experimental.pallas import tpu_sc as plsc

N = 10000
F_IN = 128
H = 128
EMB = 64
E = 320000

NC, NS = 2, 16
NW = NC * NS
CHUNK = 128
EPT = 10240
NCHUNK = EPT // CHUNK
EP = NW * EPT
NACC = 10112
RPT = NACC // NS
DRAIN = (128, 128, 128, 128, 120)

_MESH = dict(core_axis_name="c", subcore_axis_name="s")

BN_EPS = 1e-5
R = 2000
GB = N // R


NP = 10240
SPT = NP // NS
DCHUNK = 128
DNCHUNK = EPT // DCHUNK


def _sc_deg_body(dst_hbm, w_hbm, out_hbm, dst_v, w_v, stage_v, acc_sh):
    cid = lax.axis_index("c")
    sid = lax.axis_index("s")
    wid = cid * NS + sid

    def z_body(i, c):
        stage_v[pl.ds(i * 16, 16)] = jnp.zeros((16,), jnp.float32)
        return c

    lax.fori_loop(0, SPT // 16, z_body, 0)
    pltpu.sync_copy(stage_v, acc_sh.at[pl.ds(sid * SPT, SPT)])
    pltpu.sync_copy(dst_hbm.at[wid], dst_v)
    pltpu.sync_copy(w_hbm.at[wid], w_v)
    plsc.subcore_barrier()

    def chunk_body(j, c):
        pltpu.sync_copy(w_v.at[j], acc_sh.at[dst_v.at[j]], add=True)
        return c

    lax.fori_loop(0, DNCHUNK, chunk_body, 0)
    plsc.subcore_barrier()
    pltpu.sync_copy(acc_sh.at[pl.ds(sid * SPT, SPT)], stage_v)
    pltpu.sync_copy(stage_v, out_hbm.at[pl.ds(cid * NP + sid * SPT, SPT)])


def _run_deg(dst_p3, w_p3):
    call = pl.kernel(
        _sc_deg_body,
        out_type=jax.ShapeDtypeStruct((NC * NP,), jnp.float32),
        mesh=plsc.VectorSubcoreMesh(**_MESH),
        scratch_types=[
            pltpu.VMEM((DNCHUNK, DCHUNK), jnp.int32),
            pltpu.VMEM((DNCHUNK, DCHUNK), jnp.float32),
            pltpu.VMEM((SPT,), jnp.float32),
            pltpu.VMEM_SHARED((NP,), jnp.float32),
        ],
    )
    return call(dst_p3, w_p3)


def _sc_scatter_body(u_hbm, src_hbm, dst_hbm, w_hbm, out_hbm,
                     src_v, gbuf0, gbuf1,
                     dbuf0, dbuf1, dbuf2, dbuf3, wbuf0, wbuf1,
                     acc_sh, sg0, sg1, sd0, sd1, ss0, ss1):
    cid = lax.axis_index("c")
    sid = lax.axis_index("s")
    wid = cid * NS + sid
    gbufs, sgs = (gbuf0, gbuf1), (sg0, sg1)
    sss = (ss0, ss1)
    dbufs = (dbuf0, dbuf1, dbuf2, dbuf3)
    wbufs, sds = (wbuf0, wbuf1), (sd0, sd1)

    def z_body(i, c):
        for cc in range(H // 16):
            gbuf0[i, pl.ds(cc * 16, 16)] = jnp.zeros((16,), jnp.float32)
        return c

    lax.fori_loop(0, CHUNK, z_body, 0)
    off = 0
    for sz in DRAIN:
        for piece in range((sz + CHUNK - 1) // CHUNK):
            psz = min(CHUNK, sz - piece * CHUNK)
            pltpu.sync_copy(
                gbuf0.at[pl.ds(0, psz)],
                acc_sh.at[pl.ds(sid * RPT + off + piece * CHUNK, psz)])
        off += sz
    pltpu.sync_copy(src_hbm.at[wid], src_v)
    plsc.subcore_barrier()

    pltpu.async_copy(u_hbm.at[src_v.at[pl.ds(0, CHUNK)]], gbuf0, sg0)
    for b in range(2):
        pltpu.async_copy(dst_hbm.at[wid, pl.ds(b * CHUNK, CHUNK)],
                         dbufs[b], sds[b])
        pltpu.async_copy(w_hbm.at[wid, pl.ds(b * CHUNK, CHUNK)],
                         wbufs[b], sds[b])

    def quad_body(q, carry):
        for b in range(4):
            j = 4 * q + b
            p = b % 2
            gb, sg = gbufs[p], sgs[p]
            wb, sd = wbufs[p], sds[p]
            ss = sss[p]
            db = dbufs[b]
            csl = pl.ds(j * CHUNK, CHUNK)

            @pl.when(j >= 1)
            def _():
                pltpu.make_async_copy(gbufs[1 - p],
                                      acc_sh.at[dbufs[(b + 3) % 4]],
                                      sss[1 - p]).wait()

            @pl.when(j + 1 < NCHUNK)
            def _():
                psl = pl.ds((j + 1) * CHUNK, CHUNK)
                pltpu.async_copy(u_hbm.at[src_v.at[psl]],
                                 gbufs[1 - p], sgs[1 - p])

            pltpu.make_async_copy(u_hbm.at[src_v.at[csl]], gb, sg).wait()
            pltpu.make_async_copy(dst_hbm.at[wid, csl], db, sd).wait()
            pltpu.make_async_copy(w_hbm.at[wid, csl], wb, sd).wait()

            def grp_body(g, c2):
                base = g * 16
                w16 = wb[pl.ds(base, 16)]
                for r in range(16):
                    w = w16[r]
                    row = base + r
                    for c in range(H // 16):
                        sl = pl.ds(c * 16, 16)
                        gb[row, sl] = gb[row, sl] * w
                return c2

            lax.fori_loop(0, CHUNK // 16, grp_body, 0)
            pltpu.async_copy(gb, acc_sh.at[db], ss, add=True)

            @pl.when(j + 2 < NCHUNK)
            def _():
                psl = pl.ds((j + 2) * CHUNK, CHUNK)
                pltpu.async_copy(dst_hbm.at[wid, psl], dbufs[(b + 2) % 4], sd)
                pltpu.async_copy(w_hbm.at[wid, psl], wb, sd)
        return carry

    lax.fori_loop(0, NCHUNK // 4, quad_body, 0)
    pltpu.make_async_copy(gbuf1, acc_sh.at[dbuf3], ss1).wait()
    plsc.subcore_barrier()
    off = 0
    for sz in DRAIN:
        for piece in range((sz + CHUNK - 1) // CHUNK):
            psz = min(CHUNK, sz - piece * CHUNK)
            po = sid * RPT + off + piece * CHUNK
            pltpu.sync_copy(acc_sh.at[pl.ds(po, psz)],
                            gbuf0.at[pl.ds(0, psz)])
            pltpu.sync_copy(gbuf0.at[pl.ds(0, psz)],
                            out_hbm.at[cid, pl.ds(po, psz)])
        off += sz


def _run_scatter(u, src_p, dst_p, w_p):
    call = pl.kernel(
        _sc_scatter_body,
        out_type=jax.ShapeDtypeStruct((NC, NACC, H), jnp.float32),
        mesh=plsc.VectorSubcoreMesh(**_MESH),
        scratch_types=[
            pltpu.VMEM((EPT,), jnp.int32),
            pltpu.VMEM((CHUNK, H), jnp.float32),
            pltpu.VMEM((CHUNK, H), jnp.float32),
            pltpu.VMEM((CHUNK,), jnp.int32),
            pltpu.VMEM((CHUNK,), jnp.int32),
            pltpu.VMEM((CHUNK,), jnp.int32),
            pltpu.VMEM((CHUNK,), jnp.int32),
            pltpu.VMEM((CHUNK,), jnp.float32),
            pltpu.VMEM((CHUNK,), jnp.float32),
            pltpu.VMEM_SHARED((NACC, H), jnp.float32),
            pltpu.SemaphoreType.DMA,
            pltpu.SemaphoreType.DMA,
            pltpu.SemaphoreType.DMA,
            pltpu.SemaphoreType.DMA,
            pltpu.SemaphoreType.DMA,
            pltpu.SemaphoreType.DMA,
        ],
    )
    return call(u, src_p, dst_p, w_p)


def _dinv_body(d2_ref, dinv_ref):
    deg = jnp.sum(d2_ref[...], axis=0, keepdims=True) + 1.0
    dinv_ref[...] = lax.rsqrt(deg)


def _run_dinv(d2):
    return pl.pallas_call(
        _dinv_body,
        out_shape=jax.ShapeDtypeStruct((1, N), jnp.float32),
    )(d2)


def _t0_body(x_ref, winT_ref, bin_ref, w1T_ref, dinv_ref, a_ref, u_ref):
    h = jnp.dot(x_ref[...], winT_ref[...],
                preferred_element_type=jnp.float32) + bin_ref[...]
    a = jnp.dot(h, w1T_ref[...], preferred_element_type=jnp.float32)
    a_ref[...] = a
    u_ref[...] = a * dinv_ref[...]


def _run_t0(x, winT, bin_, w1T, dinv2):
    return pl.pallas_call(
        _t0_body,
        grid=(GB,),
        in_specs=[
            pl.BlockSpec((R, F_IN), lambda i: (i, 0)),
            pl.BlockSpec((F_IN, H), lambda i: (0, 0)),
            pl.BlockSpec((1, H), lambda i: (0, 0)),
            pl.BlockSpec((H, H), lambda i: (0, 0)),
            pl.BlockSpec((R, 1), lambda i: (i, 0)),
        ],
        out_specs=[
            pl.BlockSpec((R, H), lambda i: (i, 0)),
            pl.BlockSpec((R, H), lambda i: (i, 0)),
        ],
        out_shape=[
            jax.ShapeDtypeStruct((N, H), jnp.float32),
            jax.ShapeDtypeStruct((N, H), jnp.float32),
        ],
    )(x, winT, bin_, w1T, dinv2)


def _ts_body(p_ref, a_ref, b_ref, dinv_ref, out_ref, st_ref):
    ps = p_ref[0] + p_ref[1]
    dv = dinv_ref[...]
    ob = ps * dv + a_ref[...] * (dv * dv) + b_ref[...]
    out_ref[...] = ob

    @pl.when(pl.program_id(0) == 0)
    def _():
        st_ref[...] = jnp.zeros_like(st_ref)

    st_ref[...] += jnp.concatenate(
        [jnp.sum(ob, 0, keepdims=True), jnp.sum(ob * ob, 0, keepdims=True)],
        axis=0)


def _run_ts(p, a, b, dinv2):
    return pl.pallas_call(
        _ts_body,
        grid=(GB,),
        in_specs=[
            pl.BlockSpec((NC, R, H), lambda i: (0, i, 0)),
            pl.BlockSpec((R, H), lambda i: (i, 0)),
            pl.BlockSpec((1, H), lambda i: (0, 0)),
            pl.BlockSpec((R, 1), lambda i: (i, 0)),
        ],
        out_specs=[
            pl.BlockSpec((R, H), lambda i: (i, 0)),
            pl.BlockSpec((2, H), lambda i: (0, 0)),
        ],
        out_shape=[
            jax.ShapeDtypeStruct((N, H), jnp.float32),
            jax.ShapeDtypeStruct((2, H), jnp.float32),
        ],
    )(p, a, b, dinv2)


def _bn_relu(o, st, g, be):
    mean = st[0:1, :] * (1.0 / N)
    var = st[1:2, :] * (1.0 / N) - mean * mean
    return jnp.maximum((o - mean) * lax.rsqrt(var + BN_EPS) * g + be, 0.0)


def _ta_body(o_ref, st_ref, g_ref, be_ref, wT_ref, dinv_ref, a2_ref, u2_ref):
    hb = _bn_relu(o_ref[...], st_ref[...], g_ref[...], be_ref[...])
    a2 = jnp.dot(hb, wT_ref[...], preferred_element_type=jnp.float32)
    a2_ref[...] = a2
    u2_ref[...] = a2 * dinv_ref[...]


def _run_ta(o, st, g, be, wT, dinv2):
    return pl.pallas_call(
        _ta_body,
        grid=(GB,),
        in_specs=[
            pl.BlockSpec((R, H), lambda i: (i, 0)),
            pl.BlockSpec((2, H), lambda i: (0, 0)),
            pl.BlockSpec((1, H), lambda i: (0, 0)),
            pl.BlockSpec((1, H), lambda i: (0, 0)),
            pl.BlockSpec((H, H), lambda i: (0, 0)),
            pl.BlockSpec((R, 1), lambda i: (i, 0)),
        ],
        out_specs=[
            pl.BlockSpec((R, H), lambda i: (i, 0)),
            pl.BlockSpec((R, H), lambda i: (i, 0)),
        ],
        out_shape=[
            jax.ShapeDtypeStruct((N, H), jnp.float32),
            jax.ShapeDtypeStruct((N, H), jnp.float32),
        ],
    )(o, st, g, be, wT, dinv2)


def _t3_body(o_ref, st_ref, g_ref, be_ref, wn1T_ref, bn1_ref, wn2T_ref,
             bn2_ref, wc1T_ref, bc1_ref, wc2T_ref, bc2_ref,
             h_ref, node_ref, clust_ref, nsum_ref):
    hb = _bn_relu(o_ref[...], st_ref[...], g_ref[...], be_ref[...])
    h_ref[...] = hb
    z = jnp.maximum(
        jnp.dot(hb, wn1T_ref[...], preferred_element_type=jnp.float32)
        + bn1_ref[...], 0.0)
    node = jnp.dot(z, wn2T_ref[...],
                   preferred_element_type=jnp.float32) + bn2_ref[...]
    node_ref[...] = node
    c = jnp.maximum(
        jnp.dot(node, wc1T_ref[...], preferred_element_type=jnp.float32)
        + bc1_ref[...], 0.0)
    clust_ref[...] = jnp.dot(c, wc2T_ref[...],
                             preferred_element_type=jnp.float32) + bc2_ref[...]

    @pl.when(pl.program_id(0) == 0)
    def _():
        nsum_ref[...] = jnp.zeros_like(nsum_ref)

    nsum_ref[...] += jnp.sum(node, 0, keepdims=True)


def _run_t3(o, st, g, be, wn1T, bn1, wn2T, bn2, wc1T, bc1, wc2T, bc2):
    full = lambda r, c: pl.BlockSpec((r, c), lambda i: (0, 0))
    return pl.pallas_call(
        _t3_body,
        grid=(GB,),
        in_specs=[
            pl.BlockSpec((R, H), lambda i: (i, 0)),
            full(2, H), full(1, H), full(1, H),
            full(H, EMB), full(1, EMB),
            full(EMB, EMB), full(1, EMB),
            full(EMB, EMB), full(1, EMB),
            full(EMB, EMB // 2), full(1, EMB // 2),
        ],
        out_specs=[
            pl.BlockSpec((R, H), lambda i: (i, 0)),
            pl.BlockSpec((R, EMB), lambda i: (i, 0)),
            pl.BlockSpec((R, EMB // 2), lambda i: (i, 0)),
            pl.BlockSpec((1, EMB), lambda i: (0, 0)),
        ],
        out_shape=[
            jax.ShapeDtypeStruct((N, H), jnp.float32),
            jax.ShapeDtypeStruct((N, EMB), jnp.float32),
            jax.ShapeDtypeStruct((N, EMB // 2), jnp.float32),
            jax.ShapeDtypeStruct((1, EMB), jnp.float32),
        ],
    )(o, st, g, be, wn1T, bn1, wn2T, bn2, wc1T, bc1, wc2T, bc2)


def _t4_body(nsum_ref, wg1T_ref, bg1_ref, wg2T_ref, bg2_ref, graph_ref):
    m = nsum_ref[...] * (1.0 / N)
    gv = jnp.maximum(
        jnp.dot(m, wg1T_ref[...], preferred_element_type=jnp.float32)
        + bg1_ref[...], 0.0)
    graph_ref[...] = jnp.dot(gv, wg2T_ref[...],
                             preferred_element_type=jnp.float32) + bg2_ref[...]


def _run_t4(nsum, wg1T, bg1, wg2T, bg2):
    return pl.pallas_call(
        _t4_body,
        out_shape=jax.ShapeDtypeStruct((1, EMB), jnp.float32),
    )(nsum, wg1T, bg1, wg2T, bg2)


def kernel(x, edge_index, edge_weight, params):
    src = edge_index[0]
    dst = edge_index[1]
    pad = EP - E
    fill = jnp.arange(pad, dtype=src.dtype) % N
    src_p = jnp.concatenate([src, fill]).reshape(NW, EPT)
    dst_full = jnp.concatenate([dst, fill])
    w_full = jnp.concatenate([edge_weight, jnp.zeros((pad,), edge_weight.dtype)])
    dst_p = dst_full.reshape(NW, EPT)
    w_p = w_full.reshape(NW, EPT)
    dst_p3 = dst_full.reshape(NW, DNCHUNK, DCHUNK)
    w_p3 = w_full.reshape(NW, DNCHUNK, DCHUNK)

    winT = params['in_proj'][0].T
    bin_ = params['in_proj'][1].reshape(1, H)
    gcn = params['gcn']
    wT = [l['Wb'][0].T for l in gcn]
    bs = [l['Wb'][1].reshape(1, H) for l in gcn]
    gs = [l['gamma'].reshape(1, H) for l in gcn]
    bes = [l['beta'].reshape(1, H) for l in gcn]
    wn1T = params['node_emb'][0][0].T
    bn1 = params['node_emb'][0][1].reshape(1, EMB)
    wn2T = params['node_emb'][1][0].T
    bn2 = params['node_emb'][1][1].reshape(1, EMB)
    wg1T = params['graph_emb'][0][0].T
    bg1 = params['graph_emb'][0][1].reshape(1, H)
    wg2T = params['graph_emb'][1][0].T
    bg2 = params['graph_emb'][1][1].reshape(1, EMB)
    wc1T = params['clust'][0][0].T
    bc1 = params['clust'][0][1].reshape(1, EMB)
    wc2T = params['clust'][1][0].T
    bc2 = params['clust'][1][1].reshape(1, EMB // 2)

    dflat = _run_deg(dst_p3, w_p3)
    d2 = dflat.reshape(NC, NP)[:, :N]
    dinv = _run_dinv(d2)
    dinv2 = dinv.reshape(N, 1)

    a, u = _run_t0(x, winT, bin_, wT[0], dinv2)
    for i in range(3):
        p = _run_scatter(u, src_p, dst_p, w_p)
        o, st = _run_ts(p, a, bs[i], dinv2)
        if i < 2:
            a, u = _run_ta(o, st, gs[i], bes[i], wT[i + 1], dinv2)
    h, node, clust, nsum = _run_t3(o, st, gs[2], bes[2],
                                   wn1T, bn1, wn2T, bn2,
                                   wc1T, bc1, wc2T, bc2)
    graph = _run_t4(nsum, wg1T, bg1, wg2T, bg2)
    return (node, graph, clust, h)


# speedup vs baseline: 1.5018x; 1.0268x over previous
import jax
import jax.numpy as jnp
from jax import lax
from jax.experimental import pallas as pl
from jax.experimental.pallas import tpu as pltpu
from jax.experimental.pallas import tpu_sc as plsc

N = 10000
F_IN = 128
H = 128
EMB = 64
E = 320000

NC, NS = 2, 16
NW = NC * NS
CHUNK = 128
EPT = 10240
NCHUNK = EPT // CHUNK
EP = NW * EPT
NACC = 10112
RPT = NACC // NS
DRAIN = (128, 128, 128, 128, 120)

_MESH = dict(core_axis_name="c", subcore_axis_name="s")

BN_EPS = 1e-5
R = 2000
GB = N // R


NP = 10240
SPT = NP // NS
DCHUNK = 128
DNCHUNK = EPT // DCHUNK


def _sc_deg_body(dst_hbm, w_hbm, out_hbm, dst_v, w_v, stage_v, acc_sh):
    cid = lax.axis_index("c")
    sid = lax.axis_index("s")
    wid = cid * NS + sid

    def z_body(i, c):
        stage_v[pl.ds(i * 16, 16)] = jnp.zeros((16,), jnp.float32)
        return c

    lax.fori_loop(0, SPT // 16, z_body, 0)
    pltpu.sync_copy(stage_v, acc_sh.at[pl.ds(sid * SPT, SPT)])
    pltpu.sync_copy(dst_hbm.at[wid], dst_v)
    pltpu.sync_copy(w_hbm.at[wid], w_v)
    plsc.subcore_barrier()

    def chunk_body(j, c):
        pltpu.sync_copy(w_v.at[j], acc_sh.at[dst_v.at[j]], add=True)
        return c

    lax.fori_loop(0, DNCHUNK, chunk_body, 0)
    plsc.subcore_barrier()
    pltpu.sync_copy(acc_sh.at[pl.ds(sid * SPT, SPT)], stage_v)
    pltpu.sync_copy(stage_v, out_hbm.at[pl.ds(cid * NP + sid * SPT, SPT)])


def _run_deg(dst_p3, w_p3):
    call = pl.kernel(
        _sc_deg_body,
        out_type=jax.ShapeDtypeStruct((NC * NP,), jnp.float32),
        mesh=plsc.VectorSubcoreMesh(**_MESH),
        scratch_types=[
            pltpu.VMEM((DNCHUNK, DCHUNK), jnp.int32),
            pltpu.VMEM((DNCHUNK, DCHUNK), jnp.float32),
            pltpu.VMEM((SPT,), jnp.float32),
            pltpu.VMEM_SHARED((NP,), jnp.float32),
        ],
    )
    return call(dst_p3, w_p3)


def _sc_scatter_body(u_hbm, src_hbm, dst_hbm, w_hbm, out_hbm,
                     src_v, gbuf0, gbuf1,
                     dbuf0, dbuf1, dbuf2, dbuf3, wbuf0, wbuf1,
                     acc_sh, sg0, sg1, sd0, sd1, ss0, ss1):
    cid = lax.axis_index("c")
    sid = lax.axis_index("s")
    wid = cid * NS + sid
    gbufs, sgs = (gbuf0, gbuf1), (sg0, sg1)
    sss = (ss0, ss1)
    dbufs = (dbuf0, dbuf1, dbuf2, dbuf3)
    wbufs, sds = (wbuf0, wbuf1), (sd0, sd1)

    def z_body(i, c):
        for cc in range(H // 16):
            gbuf0[i, pl.ds(cc * 16, 16)] = jnp.zeros((16,), jnp.float32)
        return c

    lax.fori_loop(0, CHUNK, z_body, 0)
    off = 0
    for sz in DRAIN:
        for piece in range((sz + CHUNK - 1) // CHUNK):
            psz = min(CHUNK, sz - piece * CHUNK)
            pltpu.sync_copy(
                gbuf0.at[pl.ds(0, psz)],
                acc_sh.at[pl.ds(sid * RPT + off + piece * CHUNK, psz)])
        off += sz
    pltpu.sync_copy(src_hbm.at[wid], src_v)
    plsc.subcore_barrier()

    pltpu.async_copy(u_hbm.at[src_v.at[pl.ds(0, CHUNK)]], gbuf0, sg0)
    for b in range(2):
        pltpu.async_copy(dst_hbm.at[wid, pl.ds(b * CHUNK, CHUNK)],
                         dbufs[b], sds[b])
        pltpu.async_copy(w_hbm.at[wid, pl.ds(b * CHUNK, CHUNK)],
                         wbufs[b], sds[b])

    def quad_body(q, carry):
        for b in range(4):
            j = 4 * q + b
            p = b % 2
            gb, sg = gbufs[p], sgs[p]
            wb, sd = wbufs[p], sds[p]
            ss = sss[p]
            db = dbufs[b]
            csl = pl.ds(j * CHUNK, CHUNK)

            @pl.when(j >= 1)
            def _():
                pltpu.make_async_copy(gbufs[1 - p],
                                      acc_sh.at[dbufs[(b + 3) % 4]],
                                      sss[1 - p]).wait()

            @pl.when(j + 1 < NCHUNK)
            def _():
                psl = pl.ds((j + 1) * CHUNK, CHUNK)
                pltpu.async_copy(u_hbm.at[src_v.at[psl]],
                                 gbufs[1 - p], sgs[1 - p])

            pltpu.make_async_copy(u_hbm.at[src_v.at[csl]], gb, sg).wait()
            pltpu.make_async_copy(dst_hbm.at[wid, csl], db, sd).wait()
            pltpu.make_async_copy(w_hbm.at[wid, csl], wb, sd).wait()

            def grp_body(g, c2):
                base = g * 16
                w16 = wb[pl.ds(base, 16)]
                for r in range(16):
                    w = w16[r]
                    row = base + r
                    for c in range(H // 16):
                        sl = pl.ds(c * 16, 16)
                        gb[row, sl] = gb[row, sl] * w
                return c2

            lax.fori_loop(0, CHUNK // 16, grp_body, 0)
            pltpu.async_copy(gb, acc_sh.at[db], ss, add=True)

            @pl.when(j + 2 < NCHUNK)
            def _():
                psl = pl.ds((j + 2) * CHUNK, CHUNK)
                pltpu.async_copy(dst_hbm.at[wid, psl], dbufs[(b + 2) % 4], sd)
                pltpu.async_copy(w_hbm.at[wid, psl], wb, sd)
        return carry

    lax.fori_loop(0, NCHUNK // 4, quad_body, 0)
    pltpu.make_async_copy(gbuf1, acc_sh.at[dbuf3], ss1).wait()
    plsc.subcore_barrier()
    off = 0
    for sz in DRAIN:
        for piece in range((sz + CHUNK - 1) // CHUNK):
            psz = min(CHUNK, sz - piece * CHUNK)
            po = sid * RPT + off + piece * CHUNK
            pltpu.sync_copy(acc_sh.at[pl.ds(po, psz)],
                            gbuf0.at[pl.ds(0, psz)])
            pltpu.sync_copy(gbuf0.at[pl.ds(0, psz)],
                            out_hbm.at[cid, pl.ds(po, psz)])
        off += sz


def _run_scatter(u, src_p, dst_p, w_p):
    call = pl.kernel(
        _sc_scatter_body,
        out_type=jax.ShapeDtypeStruct((NC, NACC, H), jnp.float32),
        mesh=plsc.VectorSubcoreMesh(**_MESH),
        scratch_types=[
            pltpu.VMEM((EPT,), jnp.int32),
            pltpu.VMEM((CHUNK, H), jnp.float32),
            pltpu.VMEM((CHUNK, H), jnp.float32),
            pltpu.VMEM((CHUNK,), jnp.int32),
            pltpu.VMEM((CHUNK,), jnp.int32),
            pltpu.VMEM((CHUNK,), jnp.int32),
            pltpu.VMEM((CHUNK,), jnp.int32),
            pltpu.VMEM((CHUNK,), jnp.float32),
            pltpu.VMEM((CHUNK,), jnp.float32),
            pltpu.VMEM_SHARED((NACC, H), jnp.float32),
            pltpu.SemaphoreType.DMA,
            pltpu.SemaphoreType.DMA,
            pltpu.SemaphoreType.DMA,
            pltpu.SemaphoreType.DMA,
            pltpu.SemaphoreType.DMA,
            pltpu.SemaphoreType.DMA,
        ],
    )
    return call(u, src_p, dst_p, w_p)


def _dv_of(d2blk):
    return lax.rsqrt(jnp.sum(d2blk, axis=1, keepdims=True) + 1.0)


def _t0_body(x_ref, winT_ref, bin_ref, w1T_ref, d2_ref, a_ref, u_ref):
    h = jnp.dot(x_ref[...], winT_ref[...],
                preferred_element_type=jnp.float32) + bin_ref[...]
    a = jnp.dot(h, w1T_ref[...], preferred_element_type=jnp.float32)
    a_ref[...] = a
    u_ref[...] = a * _dv_of(d2_ref[...])


def _run_t0(x, winT, bin_, w1T, d2T):
    return pl.pallas_call(
        _t0_body,
        grid=(GB,),
        in_specs=[
            pl.BlockSpec((R, F_IN), lambda i: (i, 0)),
            pl.BlockSpec((F_IN, H), lambda i: (0, 0)),
            pl.BlockSpec((1, H), lambda i: (0, 0)),
            pl.BlockSpec((H, H), lambda i: (0, 0)),
            pl.BlockSpec((R, NC), lambda i: (i, 0)),
        ],
        out_specs=[
            pl.BlockSpec((R, H), lambda i: (i, 0)),
            pl.BlockSpec((R, H), lambda i: (i, 0)),
        ],
        out_shape=[
            jax.ShapeDtypeStruct((N, H), jnp.float32),
            jax.ShapeDtypeStruct((N, H), jnp.float32),
        ],
    )(x, winT, bin_, w1T, d2T)


def _bn_relu(o, st, g, be):
    mean = st[0:1, :] * (1.0 / N)
    var = st[1:2, :] * (1.0 / N) - mean * mean
    return jnp.maximum((o - mean) * lax.rsqrt(var + BN_EPS) * g + be, 0.0)


def _sum_stats(ob):
    return jnp.concatenate(
        [jnp.sum(ob, 0, keepdims=True), jnp.sum(ob * ob, 0, keepdims=True)],
        axis=0)


def _phase_a(p_ref, a_ref, b_ref, dv, obuf, st, i):
    ps = p_ref[0] + p_ref[1]
    ob = ps * dv + a_ref[...] * (dv * dv) + b_ref[...]
    obuf[pl.ds(pl.multiple_of(i * R, 8), R), :] = ob
    st[...] += _sum_stats(ob)


def _layer_body(p_ref, a_ref, b_ref, d2_ref, g_ref, be_ref, wT_ref,
                a2_ref, u2_ref, obuf, st):
    i = pl.program_id(0)
    dv = _dv_of(d2_ref[...])

    @pl.when(i == 0)
    def _():
        st[...] = jnp.zeros_like(st)

    @pl.when(i < GB)
    def _():
        _phase_a(p_ref, a_ref, b_ref, dv, obuf, st, i)

    @pl.when(i >= GB)
    def _():
        k = pl.multiple_of((i - GB) * R, 8)
        hb = _bn_relu(obuf[pl.ds(k, R), :], st[...], g_ref[...], be_ref[...])
        a2 = jnp.dot(hb, wT_ref[...], preferred_element_type=jnp.float32)
        a2_ref[...] = a2
        u2_ref[...] = a2 * dv


def _run_layer(p, a, b, d2T, g, be, wT):
    ia = lambda i: (jnp.minimum(i, GB - 1), 0)
    ib = lambda i: (jnp.maximum(i - GB, 0), 0)
    im = lambda i: (i % GB, 0)
    full = lambda r, c: pl.BlockSpec((r, c), lambda i: (0, 0))
    return pl.pallas_call(
        _layer_body,
        grid=(2 * GB,),
        in_specs=[
            pl.BlockSpec((NC, R, H), lambda i: (0, jnp.minimum(i, GB - 1), 0)),
            pl.BlockSpec((R, H), ia),
            full(1, H),
            pl.BlockSpec((R, NC), im),
            full(1, H), full(1, H),
            full(H, H),
        ],
        out_specs=[
            pl.BlockSpec((R, H), ib),
            pl.BlockSpec((R, H), ib),
        ],
        out_shape=[
            jax.ShapeDtypeStruct((N, H), jnp.float32),
            jax.ShapeDtypeStruct((N, H), jnp.float32),
        ],
        scratch_shapes=[
            pltpu.VMEM((N, H), jnp.float32),
            pltpu.VMEM((2, H), jnp.float32),
        ],
    )(p, a, b, d2T, g, be, wT)


def _layer3_body(p_ref, a_ref, b_ref, d2_ref, g_ref, be_ref,
                 wn1T_ref, bn1_ref, wn2T_ref, bn2_ref,
                 wc1T_ref, bc1_ref, wc2T_ref, bc2_ref,
                 h_ref, node_ref, clust_ref, nsum_ref, obuf, st):
    i = pl.program_id(0)
    dv = _dv_of(d2_ref[...])

    @pl.when(i == 0)
    def _():
        st[...] = jnp.zeros_like(st)
        nsum_ref[...] = jnp.zeros_like(nsum_ref)

    @pl.when(i < GB)
    def _():
        _phase_a(p_ref, a_ref, b_ref, dv, obuf, st, i)

    @pl.when(i >= GB)
    def _():
        k = pl.multiple_of((i - GB) * R, 8)
        hb = _bn_relu(obuf[pl.ds(k, R), :], st[...], g_ref[...], be_ref[...])
        h_ref[...] = hb
        z = jnp.maximum(
            jnp.dot(hb, wn1T_ref[...], preferred_element_type=jnp.float32)
            + bn1_ref[...], 0.0)
        node = jnp.dot(z, wn2T_ref[...],
                       preferred_element_type=jnp.float32) + bn2_ref[...]
        node_ref[...] = node
        c = jnp.maximum(
            jnp.dot(node, wc1T_ref[...], preferred_element_type=jnp.float32)
            + bc1_ref[...], 0.0)
        clust_ref[...] = jnp.dot(
            c, wc2T_ref[...], preferred_element_type=jnp.float32) + bc2_ref[...]
        nsum_ref[...] += jnp.sum(node, 0, keepdims=True)


def _run_layer3(p, a, b, d2T, g, be, wn1T, bn1, wn2T, bn2,
                wc1T, bc1, wc2T, bc2):
    ia = lambda i: (jnp.minimum(i, GB - 1), 0)
    ib = lambda i: (jnp.maximum(i - GB, 0), 0)
    im = lambda i: (i % GB, 0)
    full = lambda r, c: pl.BlockSpec((r, c), lambda i: (0, 0))
    return pl.pallas_call(
        _layer3_body,
        grid=(2 * GB,),
        in_specs=[
            pl.BlockSpec((NC, R, H), lambda i: (0, jnp.minimum(i, GB - 1), 0)),
            pl.BlockSpec((R, H), ia),
            full(1, H),
            pl.BlockSpec((R, NC), im),
            full(1, H), full(1, H),
            full(H, EMB), full(1, EMB),
            full(EMB, EMB), full(1, EMB),
            full(EMB, EMB), full(1, EMB),
            full(EMB, EMB // 2), full(1, EMB // 2),
        ],
        out_specs=[
            pl.BlockSpec((R, H), ib),
            pl.BlockSpec((R, EMB), ib),
            pl.BlockSpec((R, EMB // 2), ib),
            pl.BlockSpec((1, EMB), lambda i: (0, 0)),
        ],
        out_shape=[
            jax.ShapeDtypeStruct((N, H), jnp.float32),
            jax.ShapeDtypeStruct((N, EMB), jnp.float32),
            jax.ShapeDtypeStruct((N, EMB // 2), jnp.float32),
            jax.ShapeDtypeStruct((1, EMB), jnp.float32),
        ],
        scratch_shapes=[
            pltpu.VMEM((N, H), jnp.float32),
            pltpu.VMEM((2, H), jnp.float32),
        ],
    )(p, a, b, d2T, g, be, wn1T, bn1, wn2T, bn2, wc1T, bc1, wc2T, bc2)


def _t4_body(nsum_ref, wg1T_ref, bg1_ref, wg2T_ref, bg2_ref, graph_ref):
    m = nsum_ref[...] * (1.0 / N)
    gv = jnp.maximum(
        jnp.dot(m, wg1T_ref[...], preferred_element_type=jnp.float32)
        + bg1_ref[...], 0.0)
    graph_ref[...] = jnp.dot(gv, wg2T_ref[...],
                             preferred_element_type=jnp.float32) + bg2_ref[...]


def _run_t4(nsum, wg1T, bg1, wg2T, bg2):
    return pl.pallas_call(
        _t4_body,
        out_shape=jax.ShapeDtypeStruct((1, EMB), jnp.float32),
    )(nsum, wg1T, bg1, wg2T, bg2)


def kernel(x, edge_index, edge_weight, params):
    src = edge_index[0]
    dst = edge_index[1]
    pad = EP - E
    fill = jnp.arange(pad, dtype=src.dtype) % N
    src_p = jnp.concatenate([src, fill]).reshape(NW, EPT)
    dst_full = jnp.concatenate([dst, fill])
    w_full = jnp.concatenate([edge_weight, jnp.zeros((pad,), edge_weight.dtype)])
    dst_p = dst_full.reshape(NW, EPT)
    w_p = w_full.reshape(NW, EPT)
    dst_p3 = dst_full.reshape(NW, DNCHUNK, DCHUNK)
    w_p3 = w_full.reshape(NW, DNCHUNK, DCHUNK)

    winT = params['in_proj'][0].T
    bin_ = params['in_proj'][1].reshape(1, H)
    gcn = params['gcn']
    wT = [l['Wb'][0].T for l in gcn]
    bs = [l['Wb'][1].reshape(1, H) for l in gcn]
    gs = [l['gamma'].reshape(1, H) for l in gcn]
    bes = [l['beta'].reshape(1, H) for l in gcn]
    wn1T = params['node_emb'][0][0].T
    bn1 = params['node_emb'][0][1].reshape(1, EMB)
    wn2T = params['node_emb'][1][0].T
    bn2 = params['node_emb'][1][1].reshape(1, EMB)
    wg1T = params['graph_emb'][0][0].T
    bg1 = params['graph_emb'][0][1].reshape(1, H)
    wg2T = params['graph_emb'][1][0].T
    bg2 = params['graph_emb'][1][1].reshape(1, EMB)
    wc1T = params['clust'][0][0].T
    bc1 = params['clust'][0][1].reshape(1, EMB)
    wc2T = params['clust'][1][0].T
    bc2 = params['clust'][1][1].reshape(1, EMB // 2)

    dflat = _run_deg(dst_p3, w_p3)
    d2T = dflat.reshape(NC, NP)[:, :N].T

    a, u = _run_t0(x, winT, bin_, wT[0], d2T)
    for i in range(2):
        p = _run_scatter(u, src_p, dst_p, w_p)
        a, u = _run_layer(p, a, bs[i], d2T, gs[i], bes[i], wT[i + 1])
    p = _run_scatter(u, src_p, dst_p, w_p)
    h, node, clust, nsum = _run_layer3(p, a, bs[2], d2T, gs[2], bes[2],
                                       wn1T, bn1, wn2T, bn2,
                                       wc1T, bc1, wc2T, bc2)
    graph = _run_t4(nsum, wg1T, bg1, wg2T, bg2)
    return (node, graph, clust, h)


# speedup vs baseline: 1.5736x; 1.0478x over previous
import jax
import jax.numpy as jnp
from jax import lax
from jax.experimental import pallas as pl
from jax.experimental.pallas import tpu as pltpu
from jax.experimental.pallas import tpu_sc as plsc

N = 10000
F_IN = 128
H = 128
EMB = 64
E = 320000

NC, NS = 2, 16
NW = NC * NS
CHUNK = 80
EPT = 10240
NCHUNK = EPT // CHUNK
EP = NW * EPT
NACC = 10112
RPT = NACC // NS
DRAIN = (128, 128, 128, 128, 120)

_MESH = dict(core_axis_name="c", subcore_axis_name="s")

BN_EPS = 1e-5
R = 2000
GB = N // R


NP = 10240
SPT = NP // NS
DCHUNK = 128
DNCHUNK = EPT // DCHUNK


def _sc_deg_body(dst_hbm, w_hbm, out_hbm, dst_v, w_v, stage_v, acc_sh):
    cid = lax.axis_index("c")
    sid = lax.axis_index("s")
    wid = cid * NS + sid

    def z_body(i, c):
        stage_v[pl.ds(i * 16, 16)] = jnp.zeros((16,), jnp.float32)
        return c

    lax.fori_loop(0, SPT // 16, z_body, 0)
    pltpu.sync_copy(stage_v, acc_sh.at[pl.ds(sid * SPT, SPT)])
    pltpu.sync_copy(dst_hbm.at[wid], dst_v)
    pltpu.sync_copy(w_hbm.at[wid], w_v)
    plsc.subcore_barrier()

    def chunk_body(j, c):
        pltpu.sync_copy(w_v.at[j], acc_sh.at[dst_v.at[j]], add=True)
        return c

    lax.fori_loop(0, DNCHUNK, chunk_body, 0)
    plsc.subcore_barrier()
    pltpu.sync_copy(acc_sh.at[pl.ds(sid * SPT, SPT)], stage_v)
    pltpu.sync_copy(stage_v, out_hbm.at[pl.ds(cid * NP + sid * SPT, SPT)])


def _run_deg(dst_p3, w_p3):
    call = pl.kernel(
        _sc_deg_body,
        out_type=jax.ShapeDtypeStruct((NC * NP,), jnp.float32),
        mesh=plsc.VectorSubcoreMesh(**_MESH),
        scratch_types=[
            pltpu.VMEM((DNCHUNK, DCHUNK), jnp.int32),
            pltpu.VMEM((DNCHUNK, DCHUNK), jnp.float32),
            pltpu.VMEM((SPT,), jnp.float32),
            pltpu.VMEM_SHARED((NP,), jnp.float32),
        ],
    )
    return call(dst_p3, w_p3)


def _sc_scatter_body(u_hbm, src_hbm, dst_hbm, w_hbm, out_hbm,
                     gbuf0, gbuf1, gbuf2, gbuf3,
                     sbuf0, sbuf1, sbuf2, sbuf3,
                     dbuf0, dbuf1, dbuf2, dbuf3, wbuf0, wbuf1,
                     acc_sh,
                     sg0, sg1, sg2, sg3, ss0, ss1, ss2, ss3,
                     sr0, sr1, sr2, sr3, sd0, sd1, sd2, sd3, sw0, sw1):
    cid = lax.axis_index("c")
    sid = lax.axis_index("s")
    wid = cid * NS + sid
    gbufs = (gbuf0, gbuf1, gbuf2, gbuf3)
    sbufs = (sbuf0, sbuf1, sbuf2, sbuf3)
    dbufs = (dbuf0, dbuf1, dbuf2, dbuf3)
    wbufs = (wbuf0, wbuf1)
    sgs, sss = (sg0, sg1, sg2, sg3), (ss0, ss1, ss2, ss3)
    srs, sds = (sr0, sr1, sr2, sr3), (sd0, sd1, sd2, sd3)
    sws = (sw0, sw1)

    def z_body(i, c):
        for cc in range(H // 16):
            gbuf0[i, pl.ds(cc * 16, 16)] = jnp.zeros((16,), jnp.float32)
        return c

    lax.fori_loop(0, CHUNK, z_body, 0)
    off = 0
    for sz in DRAIN:
        for piece in range((sz + CHUNK - 1) // CHUNK):
            psz = min(CHUNK, sz - piece * CHUNK)
            pltpu.sync_copy(
                gbuf0.at[pl.ds(0, psz)],
                acc_sh.at[pl.ds(sid * RPT + off + piece * CHUNK, psz)])
        off += sz
    plsc.subcore_barrier()

    def esl(j):
        return (wid, j, pl.ds(0, CHUNK))

    for b in range(2):
        pltpu.sync_copy(src_hbm.at[esl(b)], sbufs[b])
        pltpu.async_copy(u_hbm.at[sbufs[b]], gbufs[b], sgs[b])
        pltpu.async_copy(src_hbm.at[esl(b + 2)], sbufs[b + 2],
                         srs[b + 2])
        pltpu.async_copy(dst_hbm.at[esl(b)], dbufs[b], sds[b])
        pltpu.async_copy(w_hbm.at[esl(b)], wbufs[b], sws[b])

    def quad_body(q, carry):
        for b in range(4):
            j = 4 * q + b
            gb, sg = gbufs[b], sgs[b]
            sb, ss = sbufs[b], sss[b]
            db, sd = dbufs[b], sds[b]
            wb, sw = wbufs[b % 2], sws[b % 2]
            b2 = (b + 2) % 4

            pltpu.make_async_copy(u_hbm.at[sb], gb, sg).wait()
            pltpu.make_async_copy(dst_hbm.at[esl(j)], db, sd).wait()
            pltpu.make_async_copy(w_hbm.at[esl(j)], wb, sw).wait()

            def grp_body(g, c2):
                base = g * 16
                w16 = wb[pl.ds(base, 16)]
                for r in range(16):
                    w = w16[r]
                    row = base + r
                    for c in range(H // 16):
                        sl = pl.ds(c * 16, 16)
                        gb[row, sl] = gb[row, sl] * w
                return c2

            lax.fori_loop(0, CHUNK // 16, grp_body, 0)
            pltpu.async_copy(gb, acc_sh.at[db], ss, add=True)

            @pl.when(j >= 2)
            def _():
                pltpu.make_async_copy(gbufs[b2], acc_sh.at[dbufs[b2]],
                                      sss[b2]).wait()

            @pl.when(j + 2 < NCHUNK)
            def _():
                pltpu.make_async_copy(src_hbm.at[esl(j + 2)],
                                      sbufs[b2], srs[b2]).wait()
                pltpu.async_copy(u_hbm.at[sbufs[b2]], gbufs[b2], sgs[b2])
                pltpu.async_copy(dst_hbm.at[esl(j + 2)], dbufs[b2],
                                 sds[b2])
                pltpu.async_copy(w_hbm.at[esl(j + 2)], wb, sw)

            @pl.when(j + 4 < NCHUNK)
            def _():
                pltpu.async_copy(src_hbm.at[esl(j + 4)], sb, srs[b])
        return carry

    lax.fori_loop(0, NCHUNK // 4, quad_body, 0)
    pltpu.make_async_copy(gbuf2, acc_sh.at[dbuf2], ss2).wait()
    pltpu.make_async_copy(gbuf3, acc_sh.at[dbuf3], ss3).wait()
    plsc.subcore_barrier()
    off = 0
    for sz in DRAIN:
        for piece in range((sz + CHUNK - 1) // CHUNK):
            psz = min(CHUNK, sz - piece * CHUNK)
            po = sid * RPT + off + piece * CHUNK
            pltpu.sync_copy(acc_sh.at[pl.ds(po, psz)],
                            gbuf0.at[pl.ds(0, psz)])
            pltpu.sync_copy(gbuf0.at[pl.ds(0, psz)],
                            out_hbm.at[cid, pl.ds(po, psz)])
        off += sz


def _run_scatter(u, src_p, dst_p, w_p):
    call = pl.kernel(
        _sc_scatter_body,
        out_type=jax.ShapeDtypeStruct((NC, NACC, H), jnp.float32),
        mesh=plsc.VectorSubcoreMesh(**_MESH),
        scratch_types=(
            [pltpu.VMEM((CHUNK, H), jnp.float32)] * 4
            + [pltpu.VMEM((CHUNK,), jnp.int32)] * 4
            + [pltpu.VMEM((CHUNK,), jnp.int32)] * 4
            + [pltpu.VMEM((CHUNK,), jnp.float32)] * 2
            + [pltpu.VMEM_SHARED((NACC, H), jnp.float32)]
            + [pltpu.SemaphoreType.DMA] * 18
        ),
    )
    return call(u, src_p, dst_p, w_p)


def _dv_of(d2blk):
    return lax.rsqrt(jnp.sum(d2blk, axis=1, keepdims=True) + 1.0)


def _t0_body(x_ref, winT_ref, bin_ref, w1T_ref, d2_ref, a_ref, u_ref):
    h = jnp.dot(x_ref[...], winT_ref[...],
                preferred_element_type=jnp.float32) + bin_ref[...]
    a = jnp.dot(h, w1T_ref[...], preferred_element_type=jnp.float32)
    a_ref[...] = a
    u_ref[...] = a * _dv_of(d2_ref[...])


def _run_t0(x, winT, bin_, w1T, d2T):
    return pl.pallas_call(
        _t0_body,
        grid=(GB,),
        in_specs=[
            pl.BlockSpec((R, F_IN), lambda i: (i, 0)),
            pl.BlockSpec((F_IN, H), lambda i: (0, 0)),
            pl.BlockSpec((1, H), lambda i: (0, 0)),
            pl.BlockSpec((H, H), lambda i: (0, 0)),
            pl.BlockSpec((R, NC), lambda i: (i, 0)),
        ],
        out_specs=[
            pl.BlockSpec((R, H), lambda i: (i, 0)),
            pl.BlockSpec((R, H), lambda i: (i, 0)),
        ],
        out_shape=[
            jax.ShapeDtypeStruct((N, H), jnp.float32),
            jax.ShapeDtypeStruct((N, H), jnp.float32),
        ],
    )(x, winT, bin_, w1T, d2T)


def _bn_relu(o, st, g, be):
    mean = st[0:1, :] * (1.0 / N)
    var = st[1:2, :] * (1.0 / N) - mean * mean
    return jnp.maximum((o - mean) * lax.rsqrt(var + BN_EPS) * g + be, 0.0)


def _sum_stats(ob):
    return jnp.concatenate(
        [jnp.sum(ob, 0, keepdims=True), jnp.sum(ob * ob, 0, keepdims=True)],
        axis=0)


def _phase_a(p_ref, a_ref, b_ref, dv, obuf, st, i):
    ps = p_ref[0] + p_ref[1]
    ob = ps * dv + a_ref[...] * (dv * dv) + b_ref[...]
    obuf[pl.ds(pl.multiple_of(i * R, 8), R), :] = ob
    st[...] += _sum_stats(ob)


def _layer_body(p_ref, a_ref, b_ref, d2_ref, g_ref, be_ref, wT_ref,
                a2_ref, u2_ref, obuf, st):
    i = pl.program_id(0)
    dv = _dv_of(d2_ref[...])

    @pl.when(i == 0)
    def _():
        st[...] = jnp.zeros_like(st)

    @pl.when(i < GB)
    def _():
        _phase_a(p_ref, a_ref, b_ref, dv, obuf, st, i)

    @pl.when(i >= GB)
    def _():
        k = pl.multiple_of((i - GB) * R, 8)
        hb = _bn_relu(obuf[pl.ds(k, R), :], st[...], g_ref[...], be_ref[...])
        a2 = jnp.dot(hb, wT_ref[...], preferred_element_type=jnp.float32)
        a2_ref[...] = a2
        u2_ref[...] = a2 * dv


def _run_layer(p, a, b, d2T, g, be, wT):
    ia = lambda i: (jnp.minimum(i, GB - 1), 0)
    ib = lambda i: (jnp.maximum(i - GB, 0), 0)
    im = lambda i: (i % GB, 0)
    full = lambda r, c: pl.BlockSpec((r, c), lambda i: (0, 0))
    return pl.pallas_call(
        _layer_body,
        grid=(2 * GB,),
        in_specs=[
            pl.BlockSpec((NC, R, H), lambda i: (0, jnp.minimum(i, GB - 1), 0)),
            pl.BlockSpec((R, H), ia),
            full(1, H),
            pl.BlockSpec((R, NC), im),
            full(1, H), full(1, H),
            full(H, H),
        ],
        out_specs=[
            pl.BlockSpec((R, H), ib),
            pl.BlockSpec((R, H), ib),
        ],
        out_shape=[
            jax.ShapeDtypeStruct((N, H), jnp.float32),
            jax.ShapeDtypeStruct((N, H), jnp.float32),
        ],
        scratch_shapes=[
            pltpu.VMEM((N, H), jnp.float32),
            pltpu.VMEM((2, H), jnp.float32),
        ],
    )(p, a, b, d2T, g, be, wT)


def _layer3_body(p_ref, a_ref, b_ref, d2_ref, g_ref, be_ref,
                 wn1T_ref, bn1_ref, wn2T_ref, bn2_ref,
                 wc1T_ref, bc1_ref, wc2T_ref, bc2_ref,
                 h_ref, node_ref, clust_ref, nsum_ref, obuf, st):
    i = pl.program_id(0)
    dv = _dv_of(d2_ref[...])

    @pl.when(i == 0)
    def _():
        st[...] = jnp.zeros_like(st)
        nsum_ref[...] = jnp.zeros_like(nsum_ref)

    @pl.when(i < GB)
    def _():
        _phase_a(p_ref, a_ref, b_ref, dv, obuf, st, i)

    @pl.when(i >= GB)
    def _():
        k = pl.multiple_of((i - GB) * R, 8)
        hb = _bn_relu(obuf[pl.ds(k, R), :], st[...], g_ref[...], be_ref[...])
        h_ref[...] = hb
        z = jnp.maximum(
            jnp.dot(hb, wn1T_ref[...], preferred_element_type=jnp.float32)
            + bn1_ref[...], 0.0)
        node = jnp.dot(z, wn2T_ref[...],
                       preferred_element_type=jnp.float32) + bn2_ref[...]
        node_ref[...] = node
        c = jnp.maximum(
            jnp.dot(node, wc1T_ref[...], preferred_element_type=jnp.float32)
            + bc1_ref[...], 0.0)
        clust_ref[...] = jnp.dot(
            c, wc2T_ref[...], preferred_element_type=jnp.float32) + bc2_ref[...]
        nsum_ref[...] += jnp.sum(node, 0, keepdims=True)


def _run_layer3(p, a, b, d2T, g, be, wn1T, bn1, wn2T, bn2,
                wc1T, bc1, wc2T, bc2):
    ia = lambda i: (jnp.minimum(i, GB - 1), 0)
    ib = lambda i: (jnp.maximum(i - GB, 0), 0)
    im = lambda i: (i % GB, 0)
    full = lambda r, c: pl.BlockSpec((r, c), lambda i: (0, 0))
    return pl.pallas_call(
        _layer3_body,
        grid=(2 * GB,),
        in_specs=[
            pl.BlockSpec((NC, R, H), lambda i: (0, jnp.minimum(i, GB - 1), 0)),
            pl.BlockSpec((R, H), ia),
            full(1, H),
            pl.BlockSpec((R, NC), im),
            full(1, H), full(1, H),
            full(H, EMB), full(1, EMB),
            full(EMB, EMB), full(1, EMB),
            full(EMB, EMB), full(1, EMB),
            full(EMB, EMB // 2), full(1, EMB // 2),
        ],
        out_specs=[
            pl.BlockSpec((R, H), ib),
            pl.BlockSpec((R, EMB), ib),
            pl.BlockSpec((R, EMB // 2), ib),
            pl.BlockSpec((1, EMB), lambda i: (0, 0)),
        ],
        out_shape=[
            jax.ShapeDtypeStruct((N, H), jnp.float32),
            jax.ShapeDtypeStruct((N, EMB), jnp.float32),
            jax.ShapeDtypeStruct((N, EMB // 2), jnp.float32),
            jax.ShapeDtypeStruct((1, EMB), jnp.float32),
        ],
        scratch_shapes=[
            pltpu.VMEM((N, H), jnp.float32),
            pltpu.VMEM((2, H), jnp.float32),
        ],
    )(p, a, b, d2T, g, be, wn1T, bn1, wn2T, bn2, wc1T, bc1, wc2T, bc2)


def _t4_body(nsum_ref, wg1T_ref, bg1_ref, wg2T_ref, bg2_ref, graph_ref):
    m = nsum_ref[...] * (1.0 / N)
    gv = jnp.maximum(
        jnp.dot(m, wg1T_ref[...], preferred_element_type=jnp.float32)
        + bg1_ref[...], 0.0)
    graph_ref[...] = jnp.dot(gv, wg2T_ref[...],
                             preferred_element_type=jnp.float32) + bg2_ref[...]


def _run_t4(nsum, wg1T, bg1, wg2T, bg2):
    return pl.pallas_call(
        _t4_body,
        out_shape=jax.ShapeDtypeStruct((1, EMB), jnp.float32),
    )(nsum, wg1T, bg1, wg2T, bg2)


def kernel(x, edge_index, edge_weight, params):
    src = edge_index[0]
    dst = edge_index[1]
    pad = EP - E
    fill = jnp.arange(pad, dtype=src.dtype) % N
    def chunk_pad(v):
        v3 = v.reshape(NW, NCHUNK, CHUNK)
        return jnp.pad(v3, ((0, 0), (0, 0), (0, 128 - CHUNK)))

    src_full = jnp.concatenate([src, fill])
    dst_full = jnp.concatenate([dst, fill])
    w_full = jnp.concatenate([edge_weight, jnp.zeros((pad,), edge_weight.dtype)])
    src_p = chunk_pad(src_full)
    dst_p = chunk_pad(dst_full)
    w_p = chunk_pad(w_full)
    dst_p3 = dst_full.reshape(NW, DNCHUNK, DCHUNK)
    w_p3 = w_full.reshape(NW, DNCHUNK, DCHUNK)

    winT = params['in_proj'][0].T
    bin_ = params['in_proj'][1].reshape(1, H)
    gcn = params['gcn']
    wT = [l['Wb'][0].T for l in gcn]
    bs = [l['Wb'][1].reshape(1, H) for l in gcn]
    gs = [l['gamma'].reshape(1, H) for l in gcn]
    bes = [l['beta'].reshape(1, H) for l in gcn]
    wn1T = params['node_emb'][0][0].T
    bn1 = params['node_emb'][0][1].reshape(1, EMB)
    wn2T = params['node_emb'][1][0].T
    bn2 = params['node_emb'][1][1].reshape(1, EMB)
    wg1T = params['graph_emb'][0][0].T
    bg1 = params['graph_emb'][0][1].reshape(1, H)
    wg2T = params['graph_emb'][1][0].T
    bg2 = params['graph_emb'][1][1].reshape(1, EMB)
    wc1T = params['clust'][0][0].T
    bc1 = params['clust'][0][1].reshape(1, EMB)
    wc2T = params['clust'][1][0].T
    bc2 = params['clust'][1][1].reshape(1, EMB // 2)

    dflat = _run_deg(dst_p3, w_p3)
    d2T = dflat.reshape(NC, NP)[:, :N].T

    a, u = _run_t0(x, winT, bin_, wT[0], d2T)
    for i in range(2):
        p = _run_scatter(u, src_p, dst_p, w_p)
        a, u = _run_layer(p, a, bs[i], d2T, gs[i], bes[i], wT[i + 1])
    p = _run_scatter(u, src_p, dst_p, w_p)
    h, node, clust, nsum = _run_layer3(p, a, bs[2], d2T, gs[2], bes[2],
                                       wn1T, bn1, wn2T, bn2,
                                       wc1T, bc1, wc2T, bc2)
    graph = _run_t4(nsum, wg1T, bg1, wg2T, bg2)
    return (node, graph, clust, h)


# speedup vs baseline: 1.5755x; 1.0012x over previous
import jax
import jax.numpy as jnp
from jax import lax
from jax.experimental import pallas as pl
from jax.experimental.pallas import tpu as pltpu
from jax.experimental.pallas import tpu_sc as plsc

N = 10000
F_IN = 128
H = 128
EMB = 64
E = 320000

NC, NS = 2, 16
NW = NC * NS
CHUNK = 80
EPT = 10240
NCHUNK = EPT // CHUNK
EP = NW * EPT
NACC = 10112
RPT = NACC // NS
DRAIN = (128, 128, 128, 128, 120)

_MESH = dict(core_axis_name="c", subcore_axis_name="s")

BN_EPS = 1e-5
R = 2000
GB = N // R


NP = 10240
SPT = NP // NS
DCHUNK = 128
DNCHUNK = EPT // DCHUNK


def _sc_deg_body(dst_hbm, w_hbm, out_hbm, dst_v, w_v, stage_v, acc_sh):
    cid = lax.axis_index("c")
    sid = lax.axis_index("s")
    wid = cid * NS + sid

    def z_body(i, c):
        stage_v[pl.ds(i * 16, 16)] = jnp.zeros((16,), jnp.float32)
        return c

    lax.fori_loop(0, SPT // 16, z_body, 0)
    pltpu.sync_copy(stage_v, acc_sh.at[pl.ds(sid * SPT, SPT)])
    pltpu.sync_copy(dst_hbm.at[wid], dst_v)
    pltpu.sync_copy(w_hbm.at[wid], w_v)
    plsc.subcore_barrier()

    def chunk_body(j, c):
        pltpu.sync_copy(w_v.at[j], acc_sh.at[dst_v.at[j]], add=True)
        return c

    lax.fori_loop(0, DNCHUNK, chunk_body, 0)
    plsc.subcore_barrier()
    pltpu.sync_copy(acc_sh.at[pl.ds(sid * SPT, SPT)], stage_v)
    pltpu.sync_copy(stage_v, out_hbm.at[pl.ds(cid * NP + sid * SPT, SPT)])


def _run_deg(dst_p3, w_p3):
    call = pl.kernel(
        _sc_deg_body,
        out_type=jax.ShapeDtypeStruct((NC * NP,), jnp.float32),
        mesh=plsc.VectorSubcoreMesh(**_MESH),
        scratch_types=[
            pltpu.VMEM((DNCHUNK, DCHUNK), jnp.int32),
            pltpu.VMEM((DNCHUNK, DCHUNK), jnp.float32),
            pltpu.VMEM((SPT,), jnp.float32),
            pltpu.VMEM_SHARED((NP,), jnp.float32),
        ],
    )
    return call(dst_p3, w_p3)


def _sc_scatter_body(u_hbm, src_hbm, dst_hbm, w_hbm, out_hbm,
                     gbuf0, gbuf1, gbuf2, gbuf3,
                     sbuf0, sbuf1, sbuf2, sbuf3,
                     dbuf0, dbuf1, dbuf2, dbuf3, wbuf0, wbuf1,
                     acc_sh,
                     sg0, sg1, sg2, sg3, ss0, ss1, ss2, ss3,
                     sr0, sr1, sr2, sr3, sd0, sd1, sd2, sd3, sw0, sw1):
    cid = lax.axis_index("c")
    sid = lax.axis_index("s")
    wid = cid * NS + sid
    gbufs = (gbuf0, gbuf1, gbuf2, gbuf3)
    sbufs = (sbuf0, sbuf1, sbuf2, sbuf3)
    dbufs = (dbuf0, dbuf1, dbuf2, dbuf3)
    wbufs = (wbuf0, wbuf1)
    sgs, sss = (sg0, sg1, sg2, sg3), (ss0, ss1, ss2, ss3)
    srs, sds = (sr0, sr1, sr2, sr3), (sd0, sd1, sd2, sd3)
    sws = (sw0, sw1)

    def esl(j):
        return (wid, j, pl.ds(0, CHUNK))

    for b in range(2):
        pltpu.sync_copy(src_hbm.at[esl(b)], sbufs[b])
        pltpu.async_copy(u_hbm.at[sbufs[b]], gbufs[b], sgs[b])
        pltpu.async_copy(src_hbm.at[esl(b + 2)], sbufs[b + 2],
                         srs[b + 2])
        pltpu.async_copy(dst_hbm.at[esl(b)], dbufs[b], sds[b])
        pltpu.async_copy(w_hbm.at[esl(b)], wbufs[b], sws[b])

    def z_body(i, c):
        for cc in range(H // 16):
            gbuf2[i, pl.ds(cc * 16, 16)] = jnp.zeros((16,), jnp.float32)
        return c

    lax.fori_loop(0, CHUNK, z_body, 0)
    off = 0
    for sz in DRAIN:
        for piece in range((sz + CHUNK - 1) // CHUNK):
            psz = min(CHUNK, sz - piece * CHUNK)
            pltpu.sync_copy(
                gbuf2.at[pl.ds(0, psz)],
                acc_sh.at[pl.ds(sid * RPT + off + piece * CHUNK, psz)])
        off += sz
    plsc.subcore_barrier()

    def quad_body(q, carry):
        for b in range(4):
            j = 4 * q + b
            gb, sg = gbufs[b], sgs[b]
            sb, ss = sbufs[b], sss[b]
            db, sd = dbufs[b], sds[b]
            wb, sw = wbufs[b % 2], sws[b % 2]
            b2 = (b + 2) % 4

            pltpu.make_async_copy(u_hbm.at[sb], gb, sg).wait()
            pltpu.make_async_copy(dst_hbm.at[esl(j)], db, sd).wait()
            pltpu.make_async_copy(w_hbm.at[esl(j)], wb, sw).wait()

            def grp_body(g, c2):
                base = g * 16
                w16 = wb[pl.ds(base, 16)]
                for r in range(16):
                    w = w16[r]
                    row = base + r
                    for c in range(H // 16):
                        sl = pl.ds(c * 16, 16)
                        gb[row, sl] = gb[row, sl] * w
                return c2

            lax.fori_loop(0, CHUNK // 16, grp_body, 0)
            pltpu.async_copy(gb, acc_sh.at[db], ss, add=True)

            @pl.when(j >= 2)
            def _():
                pltpu.make_async_copy(gbufs[b2], acc_sh.at[dbufs[b2]],
                                      sss[b2]).wait()

            @pl.when(j + 2 < NCHUNK)
            def _():
                pltpu.make_async_copy(src_hbm.at[esl(j + 2)],
                                      sbufs[b2], srs[b2]).wait()
                pltpu.async_copy(u_hbm.at[sbufs[b2]], gbufs[b2], sgs[b2])
                pltpu.async_copy(dst_hbm.at[esl(j + 2)], dbufs[b2],
                                 sds[b2])
                pltpu.async_copy(w_hbm.at[esl(j + 2)], wb, sw)

            @pl.when(j + 4 < NCHUNK)
            def _():
                pltpu.async_copy(src_hbm.at[esl(j + 4)], sb, srs[b])
        return carry

    lax.fori_loop(0, NCHUNK // 4, quad_body, 0)
    pltpu.make_async_copy(gbuf2, acc_sh.at[dbuf2], ss2).wait()
    pltpu.make_async_copy(gbuf3, acc_sh.at[dbuf3], ss3).wait()
    plsc.subcore_barrier()
    off = 0
    for sz in DRAIN:
        for piece in range((sz + CHUNK - 1) // CHUNK):
            psz = min(CHUNK, sz - piece * CHUNK)
            po = sid * RPT + off + piece * CHUNK
            pltpu.sync_copy(acc_sh.at[pl.ds(po, psz)],
                            gbuf0.at[pl.ds(0, psz)])
            pltpu.sync_copy(gbuf0.at[pl.ds(0, psz)],
                            out_hbm.at[cid, pl.ds(po, psz)])
        off += sz


def _run_scatter(u, src_p, dst_p, w_p):
    call = pl.kernel(
        _sc_scatter_body,
        out_type=jax.ShapeDtypeStruct((NC, NACC, H), jnp.float32),
        mesh=plsc.VectorSubcoreMesh(**_MESH),
        scratch_types=(
            [pltpu.VMEM((CHUNK, H), jnp.float32)] * 4
            + [pltpu.VMEM((CHUNK,), jnp.int32)] * 4
            + [pltpu.VMEM((CHUNK,), jnp.int32)] * 4
            + [pltpu.VMEM((CHUNK,), jnp.float32)] * 2
            + [pltpu.VMEM_SHARED((NACC, H), jnp.float32)]
            + [pltpu.SemaphoreType.DMA] * 18
        ),
    )
    return call(u, src_p, dst_p, w_p)


def _dv_of(d2blk):
    return lax.rsqrt(jnp.sum(d2blk, axis=1, keepdims=True) + 1.0)


def _t0_body(x_ref, winT_ref, bin_ref, w1T_ref, d2_ref, a_ref, u_ref):
    h = jnp.dot(x_ref[...], winT_ref[...],
                preferred_element_type=jnp.float32) + bin_ref[...]
    a = jnp.dot(h, w1T_ref[...], preferred_element_type=jnp.float32)
    a_ref[...] = a
    u_ref[...] = a * _dv_of(d2_ref[...])


def _run_t0(x, winT, bin_, w1T, d2T):
    return pl.pallas_call(
        _t0_body,
        grid=(GB,),
        in_specs=[
            pl.BlockSpec((R, F_IN), lambda i: (i, 0)),
            pl.BlockSpec((F_IN, H), lambda i: (0, 0)),
            pl.BlockSpec((1, H), lambda i: (0, 0)),
            pl.BlockSpec((H, H), lambda i: (0, 0)),
            pl.BlockSpec((R, NC), lambda i: (i, 0)),
        ],
        out_specs=[
            pl.BlockSpec((R, H), lambda i: (i, 0)),
            pl.BlockSpec((R, H), lambda i: (i, 0)),
        ],
        out_shape=[
            jax.ShapeDtypeStruct((N, H), jnp.float32),
            jax.ShapeDtypeStruct((N, H), jnp.float32),
        ],
    )(x, winT, bin_, w1T, d2T)


def _bn_relu(o, st, g, be):
    mean = st[0:1, :] * (1.0 / N)
    var = st[1:2, :] * (1.0 / N) - mean * mean
    return jnp.maximum((o - mean) * lax.rsqrt(var + BN_EPS) * g + be, 0.0)


def _sum_stats(ob):
    return jnp.concatenate(
        [jnp.sum(ob, 0, keepdims=True), jnp.sum(ob * ob, 0, keepdims=True)],
        axis=0)


def _phase_a(p_ref, a_ref, b_ref, dv, obuf, st, i):
    ps = p_ref[0] + p_ref[1]
    ob = ps * dv + a_ref[...] * (dv * dv) + b_ref[...]
    obuf[pl.ds(pl.multiple_of(i * R, 8), R), :] = ob
    st[...] += _sum_stats(ob)


def _layer_body(p_ref, a_ref, b_ref, d2_ref, g_ref, be_ref, wT_ref,
                a2_ref, u2_ref, obuf, st):
    i = pl.program_id(0)
    dv = _dv_of(d2_ref[...])

    @pl.when(i == 0)
    def _():
        st[...] = jnp.zeros_like(st)

    @pl.when(i < GB)
    def _():
        _phase_a(p_ref, a_ref, b_ref, dv, obuf, st, i)

    @pl.when(i >= GB)
    def _():
        k = pl.multiple_of((i - GB) * R, 8)
        hb = _bn_relu(obuf[pl.ds(k, R), :], st[...], g_ref[...], be_ref[...])
        a2 = jnp.dot(hb, wT_ref[...], preferred_element_type=jnp.float32)
        a2_ref[...] = a2
        u2_ref[...] = a2 * dv


def _run_layer(p, a, b, d2T, g, be, wT):
    ia = lambda i: (jnp.minimum(i, GB - 1), 0)
    ib = lambda i: (jnp.maximum(i - GB, 0), 0)
    im = lambda i: (i % GB, 0)
    full = lambda r, c: pl.BlockSpec((r, c), lambda i: (0, 0))
    return pl.pallas_call(
        _layer_body,
        grid=(2 * GB,),
        in_specs=[
            pl.BlockSpec((NC, R, H), lambda i: (0, jnp.minimum(i, GB - 1), 0)),
            pl.BlockSpec((R, H), ia),
            full(1, H),
            pl.BlockSpec((R, NC), im),
            full(1, H), full(1, H),
            full(H, H),
        ],
        out_specs=[
            pl.BlockSpec((R, H), ib),
            pl.BlockSpec((R, H), ib),
        ],
        out_shape=[
            jax.ShapeDtypeStruct((N, H), jnp.float32),
            jax.ShapeDtypeStruct((N, H), jnp.float32),
        ],
        scratch_shapes=[
            pltpu.VMEM((N, H), jnp.float32),
            pltpu.VMEM((2, H), jnp.float32),
        ],
    )(p, a, b, d2T, g, be, wT)


def _layer3_body(p_ref, a_ref, b_ref, d2_ref, g_ref, be_ref,
                 wn1T_ref, bn1_ref, wn2T_ref, bn2_ref,
                 wc1T_ref, bc1_ref, wc2T_ref, bc2_ref,
                 h_ref, node_ref, clust_ref, nsum_ref, obuf, st):
    i = pl.program_id(0)
    dv = _dv_of(d2_ref[...])

    @pl.when(i == 0)
    def _():
        st[...] = jnp.zeros_like(st)
        nsum_ref[...] = jnp.zeros_like(nsum_ref)

    @pl.when(i < GB)
    def _():
        _phase_a(p_ref, a_ref, b_ref, dv, obuf, st, i)

    @pl.when(i >= GB)
    def _():
        k = pl.multiple_of((i - GB) * R, 8)
        hb = _bn_relu(obuf[pl.ds(k, R), :], st[...], g_ref[...], be_ref[...])
        h_ref[...] = hb
        z = jnp.maximum(
            jnp.dot(hb, wn1T_ref[...], preferred_element_type=jnp.float32)
            + bn1_ref[...], 0.0)
        node = jnp.dot(z, wn2T_ref[...],
                       preferred_element_type=jnp.float32) + bn2_ref[...]
        node_ref[...] = node
        c = jnp.maximum(
            jnp.dot(node, wc1T_ref[...], preferred_element_type=jnp.float32)
            + bc1_ref[...], 0.0)
        clust_ref[...] = jnp.dot(
            c, wc2T_ref[...], preferred_element_type=jnp.float32) + bc2_ref[...]
        nsum_ref[...] += jnp.sum(node, 0, keepdims=True)


def _run_layer3(p, a, b, d2T, g, be, wn1T, bn1, wn2T, bn2,
                wc1T, bc1, wc2T, bc2):
    ia = lambda i: (jnp.minimum(i, GB - 1), 0)
    ib = lambda i: (jnp.maximum(i - GB, 0), 0)
    im = lambda i: (i % GB, 0)
    full = lambda r, c: pl.BlockSpec((r, c), lambda i: (0, 0))
    return pl.pallas_call(
        _layer3_body,
        grid=(2 * GB,),
        in_specs=[
            pl.BlockSpec((NC, R, H), lambda i: (0, jnp.minimum(i, GB - 1), 0)),
            pl.BlockSpec((R, H), ia),
            full(1, H),
            pl.BlockSpec((R, NC), im),
            full(1, H), full(1, H),
            full(H, EMB), full(1, EMB),
            full(EMB, EMB), full(1, EMB),
            full(EMB, EMB), full(1, EMB),
            full(EMB, EMB // 2), full(1, EMB // 2),
        ],
        out_specs=[
            pl.BlockSpec((R, H), ib),
            pl.BlockSpec((R, EMB), ib),
            pl.BlockSpec((R, EMB // 2), ib),
            pl.BlockSpec((1, EMB), lambda i: (0, 0)),
        ],
        out_shape=[
            jax.ShapeDtypeStruct((N, H), jnp.float32),
            jax.ShapeDtypeStruct((N, EMB), jnp.float32),
            jax.ShapeDtypeStruct((N, EMB // 2), jnp.float32),
            jax.ShapeDtypeStruct((1, EMB), jnp.float32),
        ],
        scratch_shapes=[
            pltpu.VMEM((N, H), jnp.float32),
            pltpu.VMEM((2, H), jnp.float32),
        ],
    )(p, a, b, d2T, g, be, wn1T, bn1, wn2T, bn2, wc1T, bc1, wc2T, bc2)


def _t4_body(nsum_ref, wg1T_ref, bg1_ref, wg2T_ref, bg2_ref, graph_ref):
    m = nsum_ref[...] * (1.0 / N)
    gv = jnp.maximum(
        jnp.dot(m, wg1T_ref[...], preferred_element_type=jnp.float32)
        + bg1_ref[...], 0.0)
    graph_ref[...] = jnp.dot(gv, wg2T_ref[...],
                             preferred_element_type=jnp.float32) + bg2_ref[...]


def _run_t4(nsum, wg1T, bg1, wg2T, bg2):
    return pl.pallas_call(
        _t4_body,
        out_shape=jax.ShapeDtypeStruct((1, EMB), jnp.float32),
    )(nsum, wg1T, bg1, wg2T, bg2)


def kernel(x, edge_index, edge_weight, params):
    src = edge_index[0]
    dst = edge_index[1]
    pad = EP - E
    fill = jnp.arange(pad, dtype=src.dtype) % N
    def chunk_pad(v):
        v3 = v.reshape(NW, NCHUNK, CHUNK)
        return jnp.pad(v3, ((0, 0), (0, 0), (0, 128 - CHUNK)))

    src_full = jnp.concatenate([src, fill])
    dst_full = jnp.concatenate([dst, fill])
    w_full = jnp.concatenate([edge_weight, jnp.zeros((pad,), edge_weight.dtype)])
    src_p = chunk_pad(src_full)
    dst_p = chunk_pad(dst_full)
    w_p = chunk_pad(w_full)
    dst_p3 = dst_full.reshape(NW, DNCHUNK, DCHUNK)
    w_p3 = w_full.reshape(NW, DNCHUNK, DCHUNK)

    winT = params['in_proj'][0].T
    bin_ = params['in_proj'][1].reshape(1, H)
    gcn = params['gcn']
    wT = [l['Wb'][0].T for l in gcn]
    bs = [l['Wb'][1].reshape(1, H) for l in gcn]
    gs = [l['gamma'].reshape(1, H) for l in gcn]
    bes = [l['beta'].reshape(1, H) for l in gcn]
    wn1T = params['node_emb'][0][0].T
    bn1 = params['node_emb'][0][1].reshape(1, EMB)
    wn2T = params['node_emb'][1][0].T
    bn2 = params['node_emb'][1][1].reshape(1, EMB)
    wg1T = params['graph_emb'][0][0].T
    bg1 = params['graph_emb'][0][1].reshape(1, H)
    wg2T = params['graph_emb'][1][0].T
    bg2 = params['graph_emb'][1][1].reshape(1, EMB)
    wc1T = params['clust'][0][0].T
    bc1 = params['clust'][0][1].reshape(1, EMB)
    wc2T = params['clust'][1][0].T
    bc2 = params['clust'][1][1].reshape(1, EMB // 2)

    dflat = _run_deg(dst_p3, w_p3)
    d2T = dflat.reshape(NC, NP)[:, :N].T

    a, u = _run_t0(x, winT, bin_, wT[0], d2T)
    for i in range(2):
        p = _run_scatter(u, src_p, dst_p, w_p)
        a, u = _run_layer(p, a, bs[i], d2T, gs[i], bes[i], wT[i + 1])
    p = _run_scatter(u, src_p, dst_p, w_p)
    h, node, clust, nsum = _run_layer3(p, a, bs[2], d2T, gs[2], bes[2],
                                       wn1T, bn1, wn2T, bn2,
                                       wc1T, bc1, wc2T, bc2)
    graph = _run_t4(nsum, wg1T, bg1, wg2T, bg2)
    return (node, graph, clust, h)
